# baseline (speedup 1.0000x reference)
import functools

import jax
import jax.numpy as jnp
from jax import lax
from jax.experimental import pallas as pl
from jax.experimental.pallas import tpu as pltpu
from jax.experimental.pallas import tpu_sc as plsc

N = 10000
E = 320000
H = 128
ED = 4
NG = 64
NC = 2
NS = 16
NW = NC * NS
C = 128
NCHUNK = E // C
NT_BASE = NCHUNK // NW
NT_REM = NCHUNK % NW
CD = 80
NTD = (E // CD) // NW
NP = 10240
RPS = NP // NS
RB = 1024
REB = 4000

_INTERPRET = False


def _in_proj_body(x_ref, w_ref, b_ref, wa_ref, wb_ref, h_ref, a_ref, b2_ref):
    h = jnp.dot(x_ref[...], w_ref[...], preferred_element_type=jnp.float32)
    h = jnp.maximum(h + b_ref[0:1, :], 0.0)
    h_ref[...] = h
    a_ref[...] = (jnp.dot(h, wa_ref[...], preferred_element_type=jnp.float32)
                  + b_ref[1:2, :])
    b2_ref[...] = jnp.dot(h, wb_ref[...], preferred_element_type=jnp.float32)


def _in_proj(x, in_w, in_b8, w1a, w1b):
    wspec = pl.BlockSpec((H, H), lambda i: (0, 0))
    bspec = pl.BlockSpec((8, H), lambda i: (0, 0))
    nspec = pl.BlockSpec((RB, H), lambda i: (i, 0))
    return pl.pallas_call(
        _in_proj_body,
        grid=(NP // RB,),
        in_specs=[nspec, wspec, bspec, wspec, wspec],
        out_specs=[nspec, nspec, nspec],
        out_shape=[jax.ShapeDtypeStruct((NP, H), jnp.float32)] * 3,
        interpret=_INTERPRET,
    )(x, in_w, in_b8, w1a, w1b)


def _make_update_body(first, has_next):
    def body(h_ref, s0_ref, s1_ref, d0_ref, d1_ref, mw2_ref, uw1a_ref,
             uw1b_ref, uw2_ref, vec_ref, *rest):
        h = h_ref[...]
        s = s0_ref[...] + s1_ref[...]
        deg = (d0_ref[...] + d1_ref[...]).astype(jnp.float32)[:, 0:1]
        aggr = jnp.dot(s, mw2_ref[...], preferred_element_type=jnp.float32)
        aggr = aggr + deg * vec_ref[0:1, :]
        t = jnp.dot(h, uw1a_ref[...], preferred_element_type=jnp.float32)
        t = t + jnp.dot(aggr, uw1b_ref[...], preferred_element_type=jnp.float32)
        t = jnp.maximum(t + vec_ref[1:2, :], 0.0)
        u = jnp.dot(t, uw2_ref[...], preferred_element_type=jnp.float32)
        u = u + vec_ref[2:3, :]
        mu = jnp.mean(u, axis=1, keepdims=True)
        var = jnp.mean((u - mu) ** 2, axis=1, keepdims=True)
        un = (u - mu) * lax.rsqrt(var + 1e-5) * vec_ref[3:4, :] + vec_ref[4:5, :]
        hn = jnp.maximum(un, 0.0)
        if not first:
            hn = hn + h
        if has_next:
            wa_ref, wb_ref, hn_ref, a_ref, b_ref = rest
            hn_ref[...] = hn
            a_ref[...] = (jnp.dot(hn, wa_ref[...],
                                  preferred_element_type=jnp.float32)
                          + vec_ref[5:6, :])
            b_ref[...] = jnp.dot(hn, wb_ref[...], preferred_element_type=jnp.float32)
        else:
            (hn_ref,) = rest
            hn_ref[...] = hn
    return body


def _update(first, has_next, h, s0, s1, d0, d1, mw2, uw1a, uw1b, uw2, vec,
            wa_next=None, wb_next=None):
    nspec = pl.BlockSpec((RB, H), lambda i: (i, 0))
    wspec = pl.BlockSpec((H, H), lambda i: (0, 0))
    vspec = pl.BlockSpec((8, H), lambda i: (0, 0))
    in_specs = [nspec, nspec, nspec, nspec, nspec, wspec, wspec, wspec,
                wspec, vspec]
    args = [h, s0, s1, d0, d1, mw2, uw1a, uw1b, uw2, vec]
    if has_next:
        in_specs += [wspec, wspec]
        args += [wa_next, wb_next]
        out_specs = [nspec, nspec, nspec]
        out_shape = [jax.ShapeDtypeStruct((NP, H), jnp.float32)] * 3
    else:
        out_specs = [nspec]
        out_shape = [jax.ShapeDtypeStruct((NP, H), jnp.float32)]
    return pl.pallas_call(
        _make_update_body(first, has_next),
        grid=(NP // RB,),
        in_specs=in_specs,
        out_specs=out_specs,
        out_shape=out_shape,
        interpret=_INTERPRET,
    )(*args)


def _pool_body(h_ref, bid_ref, w_ref, b_ref, o_ref, acc_s, acc_c):
    i = pl.program_id(0)

    @pl.when(i == 0)
    def _():
        acc_s[...] = jnp.zeros_like(acc_s)
        acc_c[...] = jnp.zeros_like(acc_c)

    h = h_ref[...]
    lanes = lax.broadcasted_iota(jnp.int32, (RB, H), 1)
    onehot = (bid_ref[...] == lanes).astype(jnp.float32)
    dn = (((0,), (0,)), ((), ()))
    acc_s[...] += lax.dot_general(onehot, h, dn, preferred_element_type=jnp.float32)
    acc_c[...] += lax.dot_general(onehot, jnp.ones_like(h), dn,
                                  preferred_element_type=jnp.float32)

    @pl.when(i == pl.num_programs(0) - 1)
    def _():
        rep = acc_s[...] / jnp.maximum(acc_c[...], 1.0)
        o_ref[...] = (
            jnp.dot(rep[:NG, :], w_ref[...], preferred_element_type=jnp.float32)
            + b_ref[0:1, :]
        )


def _pool(h, bid_rep, out_w, out_b8):
    return pl.pallas_call(
        _pool_body,
        grid=(NP // RB,),
        in_specs=[
            pl.BlockSpec((RB, H), lambda i: (i, 0)),
            pl.BlockSpec((RB, H), lambda i: (i, 0)),
            pl.BlockSpec((H, H), lambda i: (0, 0)),
            pl.BlockSpec((8, H), lambda i: (0, 0)),
        ],
        out_specs=pl.BlockSpec((NG, H), lambda i: (0, 0)),
        out_shape=jax.ShapeDtypeStruct((NG, H), jnp.float32),
        scratch_shapes=[
            pltpu.VMEM((H, H), jnp.float32),
            pltpu.VMEM((H, H), jnp.float32),
        ],
        interpret=_INTERPRET,
    )(h, bid_rep, out_w, out_b8)


def _sc_mesh():
    return plsc.VectorSubcoreMesh(
        core_axis_name="c", subcore_axis_name="s",
        num_cores=NC, num_subcores=NS)


def _zero_f32(buf, rows):
    def zbody(r, _):
        for k8 in range(H // 16):
            buf[r, pl.ds(k8 * 16, 16)] = jnp.zeros((16,), jnp.float32)
        return 0
    lax.fori_loop(0, rows, zbody, 0)


def _make_edge_sc():
    @functools.partial(
        pl.kernel,
        mesh=_sc_mesh(),
        out_type=jax.ShapeDtypeStruct((2 * NP, H), jnp.float32),
        scratch_types=[
            pltpu.VMEM_SHARED((NP, H), jnp.float32),
            pltpu.VMEM((C,), jnp.int32),
            pltpu.VMEM((C,), jnp.int32),
            pltpu.VMEM((C * ED,), jnp.float32),
            pltpu.VMEM((C, H), jnp.float32),
            pltpu.VMEM((C, H), jnp.float32),
            pltpu.VMEM((ED, H), jnp.float32),
            pltpu.SemaphoreType.DMA,
            pltpu.SemaphoreType.DMA,
        ],
        interpret=_INTERPRET,
    )
    def k(a_hbm, b_hbm, attr_hbm, w_hbm, row_hbm, col_hbm, out_hbm,
          s_sh, idx_r, idx_c, buf_t, buf_a, buf_b, wv, sem_a, sem_b):
        cid = lax.axis_index("c")
        sid = lax.axis_index("s")
        wid = sid * NC + cid

        pltpu.sync_copy(w_hbm, wv)
        _zero_f32(buf_a, C)
        rbase = sid * RPS
        for t in range(RPS // C):
            pltpu.sync_copy(buf_a, s_sh.at[pl.ds(rbase + t * C, C)])
        plsc.subcore_barrier()

        nt = NT_BASE + jnp.where(wid < NT_REM, 1, 0)

        def chunk_body(t, _):
            ebase = (t * NW + wid) * C
            pltpu.sync_copy(row_hbm.at[pl.ds(ebase, C)], idx_r)
            pltpu.sync_copy(col_hbm.at[pl.ds(ebase, C)], idx_c)
            cp_a = pltpu.async_copy(a_hbm.at[idx_r], buf_a, sem_a)
            cp_b = pltpu.async_copy(b_hbm.at[idx_c], buf_b, sem_b)
            pltpu.sync_copy(attr_hbm.at[pl.ds(ebase * ED, C * ED)], buf_t)
            cp_a.wait()
            cp_b.wait()

            def comp(g, _):
                va = buf_t[pl.ds(g * 16, 16)]
                for j in range(4):
                    r = g * 4 + j
                    bc = [
                        va.at[jnp.full((16,), 4 * j + kk, jnp.int32)]
                        .get(mode="promise_in_bounds")
                        for kk in range(ED)
                    ]
                    for k8 in range(H // 16):
                        sl = pl.ds(k8 * 16, 16)
                        v = buf_a[r, sl] + buf_b[r, sl]
                        for kk in range(ED):
                            v = v + bc[kk] * wv[kk, sl]
                        buf_a[r, sl] = jnp.maximum(v, 0.0)
                return 0

            lax.fori_loop(0, C // 4, comp, 0)
            pltpu.sync_copy(buf_a, s_sh.at[idx_r], add=True)
            return 0

        lax.fori_loop(0, nt, chunk_body, 0)
        plsc.subcore_barrier()

        obase = cid * NP + rbase
        for t in range(RPS // C):
            pltpu.sync_copy(s_sh.at[pl.ds(rbase + t * C, C)],
                            out_hbm.at[pl.ds(obase + t * C, C)])

    return k


def _make_deg_sc():
    @functools.partial(
        pl.kernel,
        mesh=_sc_mesh(),
        out_type=jax.ShapeDtypeStruct((2 * NP, H), jnp.float32),
        scratch_types=[
            pltpu.VMEM_SHARED((NP, H), jnp.float32),
            pltpu.VMEM((CD,), jnp.int32),
            pltpu.VMEM((CD, H), jnp.float32),
        ],
        interpret=_INTERPRET,
    )
    def k(row_hbm, out_hbm, s_sh, idx_r, ones_b):
        cid = lax.axis_index("c")
        sid = lax.axis_index("s")
        wid = sid * NC + cid

        _zero_f32(ones_b, CD)
        rbase = sid * RPS
        for t in range(RPS // CD):
            pltpu.sync_copy(ones_b, s_sh.at[pl.ds(rbase + t * CD, CD)])

        def obody(r, _):
            for k8 in range(H // 16):
                ones_b[r, pl.ds(k8 * 16, 16)] = jnp.ones((16,), jnp.float32)
            return 0
        lax.fori_loop(0, CD, obody, 0)
        plsc.subcore_barrier()

        def chunk_body(t, _):
            ebase = (t * NW + wid) * CD
            pltpu.sync_copy(row_hbm.at[pl.ds(ebase, CD)], idx_r)
            pltpu.sync_copy(ones_b, s_sh.at[idx_r], add=True)
            return 0

        lax.fori_loop(0, NTD, chunk_body, 0)
        plsc.subcore_barrier()

        obase = cid * NP + rbase
        for t in range(RPS // CD):
            pltpu.sync_copy(s_sh.at[pl.ds(rbase + t * CD, CD)],
                            out_hbm.at[pl.ds(obase + t * CD, CD)])

    return k


def _pad8(v):
    return jnp.broadcast_to(v[None, :], (8, H))


def kernel(x, edge_index, edge_attr, batch, params):
    row = edge_index[0]
    col = edge_index[1]
    layers = params["layers"]

    w1a = [p["mW1"][:H] for p in layers]
    w1b = [p["mW1"][H:2 * H] for p in layers]
    wea = [p["mW1"][2 * H:] for p in layers]
    vecs = [jnp.concatenate(
        [jnp.stack([p["mb2"], p["ub1"], p["ub2"], p["ln_g"], p["ln_b"]]),
         jnp.stack([layers[i + 1]["mb1"]]) if i < 2
         else jnp.zeros((1, H), jnp.float32),
         jnp.zeros((2, H), jnp.float32)], axis=0)
        for i, p in enumerate(layers)]
    uw1a = [p["uW1"][:H] for p in layers]
    uw1b = [p["uW1"][H:] for p in layers]
    x_pad = jnp.pad(x, ((0, NP - N), (0, 0)))
    batch_pad = jnp.pad(batch.astype(jnp.int32), (0, NP - N),
                        constant_values=127)
    attr_flat = edge_attr.reshape(E * ED)
    in_b8 = jnp.concatenate(
        [jnp.stack([params["in_b"], layers[0]["mb1"]]),
         jnp.zeros((6, H), jnp.float32)], axis=0)

    h, a_cur, b_cur = _in_proj(x_pad, params["in_W"], in_b8,
                               w1a[0], w1b[0])

    sc_edge = _make_edge_sc()
    deg_all = _make_deg_sc()(row)
    d0, d1 = deg_all[:NP], deg_all[NP:]

    for i in range(3):
        s_all = sc_edge(a_cur, b_cur, attr_flat, wea[i], row, col)
        s0, s1 = s_all[:NP], s_all[NP:]
        p = layers[i]
        if i < 2:
            h, a_cur, b_cur = _update(
                i == 0, True, h, s0, s1, d0, d1,
                p["mW2"], uw1a[i], uw1b[i], p["uW2"], vecs[i],
                w1a[i + 1], w1b[i + 1])
        else:
            (h,) = _update(
                False, False, h, s0, s1, d0, d1,
                p["mW2"], uw1a[i], uw1b[i], p["uW2"], vecs[i])

    bid_rep = jnp.broadcast_to(batch_pad[:, None], (NP, H))
    return _pool(h, bid_rep, params["out_W"], _pad8(params["out_b"]))


# speedup vs baseline: 1.8169x; 1.8169x over previous
import functools

import jax
import jax.numpy as jnp
from jax import lax
from jax.experimental import pallas as pl
from jax.experimental.pallas import tpu as pltpu
from jax.experimental.pallas import tpu_sc as plsc

N = 10000
E = 320000
H = 128
ED = 4
NG = 64
NC = 2
NS = 16
NW = NC * NS
C = 80
NT = (E // C) // NW
CD = 80
NTD = (E // CD) // NW
NP = 10240
RPS = NP // NS
RB = 1024
REB = 4000

_INTERPRET = False


def _in_proj_body(x_ref, w_ref, b_ref, wa_ref, wb_ref, h_ref, a_ref, b2_ref):
    h = jnp.dot(x_ref[...], w_ref[...], preferred_element_type=jnp.float32)
    h = jnp.maximum(h + b_ref[0:1, :], 0.0)
    h_ref[...] = h
    a_ref[...] = jnp.dot(h, wa_ref[...], preferred_element_type=jnp.float32)
    b2_ref[...] = jnp.dot(h, wb_ref[...], preferred_element_type=jnp.float32)


def _in_proj(x, in_w, in_b8, w1a, w1b):
    wspec = pl.BlockSpec((H, H), lambda i: (0, 0))
    bspec = pl.BlockSpec((8, H), lambda i: (0, 0))
    nspec = pl.BlockSpec((RB, H), lambda i: (i, 0))
    return pl.pallas_call(
        _in_proj_body,
        grid=(NP // RB,),
        in_specs=[nspec, wspec, bspec, wspec, wspec],
        out_specs=[nspec, nspec, nspec],
        out_shape=[jax.ShapeDtypeStruct((NP, H), jnp.float32)] * 3,
        interpret=_INTERPRET,
    )(x, in_w, in_b8, w1a, w1b)


def _edge_pre_body(attr_ref, w_ref, b_ref, e1_ref, e2_ref, e3_ref):
    a = attr_ref[...]
    outs = [e1_ref, e2_ref, e3_ref]
    for i in range(3):
        w = w_ref[4 * i:4 * (i + 1), :]
        outs[i][...] = (
            jnp.dot(a, w, preferred_element_type=jnp.float32) + b_ref[i:i + 1, :]
        )


def _edge_pre(edge_attr, wea, mb1s):
    espec = pl.BlockSpec((REB, H), lambda i: (i, 0))
    return pl.pallas_call(
        _edge_pre_body,
        grid=(E // REB,),
        in_specs=[
            pl.BlockSpec((REB, ED), lambda i: (i, 0)),
            pl.BlockSpec((16, H), lambda i: (0, 0)),
            pl.BlockSpec((8, H), lambda i: (0, 0)),
        ],
        out_specs=[espec, espec, espec],
        out_shape=[jax.ShapeDtypeStruct((E, H), jnp.float32)] * 3,
        interpret=_INTERPRET,
    )(edge_attr, wea, mb1s)


def _make_update_body(first, has_next):
    def body(h_ref, s0_ref, s1_ref, d0_ref, d1_ref, mw2_ref, uw1a_ref,
             uw1b_ref, uw2_ref, vec_ref, *rest):
        h = h_ref[...]
        s = s0_ref[...] + s1_ref[...]
        deg = (d0_ref[...] + d1_ref[...]).astype(jnp.float32)[:, 0:1]
        aggr = jnp.dot(s, mw2_ref[...], preferred_element_type=jnp.float32)
        aggr = aggr + deg * vec_ref[0:1, :]
        t = jnp.dot(h, uw1a_ref[...], preferred_element_type=jnp.float32)
        t = t + jnp.dot(aggr, uw1b_ref[...], preferred_element_type=jnp.float32)
        t = jnp.maximum(t + vec_ref[1:2, :], 0.0)
        u = jnp.dot(t, uw2_ref[...], preferred_element_type=jnp.float32)
        u = u + vec_ref[2:3, :]
        mu = jnp.mean(u, axis=1, keepdims=True)
        var = jnp.mean((u - mu) ** 2, axis=1, keepdims=True)
        un = (u - mu) * lax.rsqrt(var + 1e-5) * vec_ref[3:4, :] + vec_ref[4:5, :]
        hn = jnp.maximum(un, 0.0)
        if not first:
            hn = hn + h
        if has_next:
            wa_ref, wb_ref, hn_ref, a_ref, b_ref = rest
            hn_ref[...] = hn
            a_ref[...] = jnp.dot(hn, wa_ref[...], preferred_element_type=jnp.float32)
            b_ref[...] = jnp.dot(hn, wb_ref[...], preferred_element_type=jnp.float32)
        else:
            (hn_ref,) = rest
            hn_ref[...] = hn
    return body


def _update(first, has_next, h, s0, s1, d0, d1, mw2, uw1a, uw1b, uw2, vec,
            wa_next=None, wb_next=None):
    nspec = pl.BlockSpec((RB, H), lambda i: (i, 0))
    wspec = pl.BlockSpec((H, H), lambda i: (0, 0))
    vspec = pl.BlockSpec((8, H), lambda i: (0, 0))
    in_specs = [nspec, nspec, nspec, nspec, nspec, wspec, wspec, wspec,
                wspec, vspec]
    args = [h, s0, s1, d0, d1, mw2, uw1a, uw1b, uw2, vec]
    if has_next:
        in_specs += [wspec, wspec]
        args += [wa_next, wb_next]
        out_specs = [nspec, nspec, nspec]
        out_shape = [jax.ShapeDtypeStruct((NP, H), jnp.float32)] * 3
    else:
        out_specs = [nspec]
        out_shape = [jax.ShapeDtypeStruct((NP, H), jnp.float32)]
    return pl.pallas_call(
        _make_update_body(first, has_next),
        grid=(NP // RB,),
        in_specs=in_specs,
        out_specs=out_specs,
        out_shape=out_shape,
        interpret=_INTERPRET,
    )(*args)


def _pool_body(h_ref, bid_ref, w_ref, b_ref, o_ref, acc_s, acc_c):
    i = pl.program_id(0)

    @pl.when(i == 0)
    def _():
        acc_s[...] = jnp.zeros_like(acc_s)
        acc_c[...] = jnp.zeros_like(acc_c)

    h = h_ref[...]
    lanes = lax.broadcasted_iota(jnp.int32, (RB, H), 1)
    onehot = (bid_ref[...] == lanes).astype(jnp.float32)
    dn = (((0,), (0,)), ((), ()))
    acc_s[...] += lax.dot_general(onehot, h, dn, preferred_element_type=jnp.float32)
    acc_c[...] += lax.dot_general(onehot, jnp.ones_like(h), dn,
                                  preferred_element_type=jnp.float32)

    @pl.when(i == pl.num_programs(0) - 1)
    def _():
        rep = acc_s[...] / jnp.maximum(acc_c[...], 1.0)
        o_ref[...] = (
            jnp.dot(rep[:NG, :], w_ref[...], preferred_element_type=jnp.float32)
            + b_ref[0:1, :]
        )


def _pool(h, bid_rep, out_w, out_b8):
    return pl.pallas_call(
        _pool_body,
        grid=(NP // RB,),
        in_specs=[
            pl.BlockSpec((RB, H), lambda i: (i, 0)),
            pl.BlockSpec((RB, H), lambda i: (i, 0)),
            pl.BlockSpec((H, H), lambda i: (0, 0)),
            pl.BlockSpec((8, H), lambda i: (0, 0)),
        ],
        out_specs=pl.BlockSpec((NG, H), lambda i: (0, 0)),
        out_shape=jax.ShapeDtypeStruct((NG, H), jnp.float32),
        scratch_shapes=[
            pltpu.VMEM((H, H), jnp.float32),
            pltpu.VMEM((H, H), jnp.float32),
        ],
        interpret=_INTERPRET,
    )(h, bid_rep, out_w, out_b8)


def _sc_mesh():
    return plsc.VectorSubcoreMesh(
        core_axis_name="c", subcore_axis_name="s",
        num_cores=NC, num_subcores=NS)


def _zero_f32(buf, rows):
    def zbody(r, _):
        for k8 in range(H // 16):
            buf[r, pl.ds(k8 * 16, 16)] = jnp.zeros((16,), jnp.float32)
        return 0
    lax.fori_loop(0, rows, zbody, 0)


def _make_edge_sc():
    @functools.partial(
        pl.kernel,
        mesh=_sc_mesh(),
        out_type=jax.ShapeDtypeStruct((2 * NP, H), jnp.float32),
        scratch_types=[
            pltpu.VMEM_SHARED((NP, H), jnp.float32),
            [pltpu.VMEM((C,), jnp.int32)] * 2,
            pltpu.VMEM((C,), jnp.int32),
            pltpu.VMEM((C, H), jnp.float32),
            pltpu.VMEM((C, H), jnp.float32),
            pltpu.VMEM((C, H), jnp.float32),
            pltpu.VMEM((C, H), jnp.float32),
            pltpu.SemaphoreType.DMA,
            pltpu.SemaphoreType.DMA,
            pltpu.SemaphoreType.DMA,
        ],
        interpret=_INTERPRET,
    )
    def k(a_hbm, b_hbm, e_hbm, row_hbm, col_hbm, out_hbm,
          s_sh, idx_r, idx_c, buf_a, buf_b, buf_e, buf_o,
          sem_a, sem_b, sem_s):
        cid = lax.axis_index("c")
        sid = lax.axis_index("s")
        wid = sid * NC + cid

        _zero_f32(buf_o, C)
        rbase = sid * RPS
        for t in range(RPS // C):
            pltpu.sync_copy(buf_o, s_sh.at[pl.ds(rbase + t * C, C)])
        plsc.subcore_barrier()

        def wait_scatter(par):
            pltpu.make_async_copy(buf_o, s_sh.at[idx_r[par]], sem_s).wait()

        def do_chunk(t, par):
            ebase = (t * NW + wid) * C
            pltpu.sync_copy(row_hbm.at[pl.ds(ebase, C)], idx_r[par])
            pltpu.sync_copy(col_hbm.at[pl.ds(ebase, C)], idx_c)
            cp_a = pltpu.async_copy(a_hbm.at[idx_r[par]], buf_a, sem_a)
            cp_b = pltpu.async_copy(b_hbm.at[idx_c], buf_b, sem_b)
            pltpu.sync_copy(e_hbm.at[pl.ds(ebase, C)], buf_e)
            cp_a.wait()
            cp_b.wait()

            @pl.when(t > 0)
            def _():
                wait_scatter(1 - par)

            def comp(r, _):
                for k8 in range(H // 16):
                    sl = pl.ds(k8 * 16, 16)
                    v = buf_a[r, sl] + buf_b[r, sl] + buf_e[r, sl]
                    buf_o[r, sl] = jnp.maximum(v, 0.0)
                return 0

            lax.fori_loop(0, C, comp, 0)
            pltpu.async_copy(buf_o, s_sh.at[idx_r[par]], sem_s, add=True)

        def pair_body(j, _):
            for par in range(2):
                t = j * 2 + par
                do_chunk(t, par)
            return 0

        lax.fori_loop(0, NT // 2, pair_body, 0)
        do_chunk(NT - 1, (NT - 1) % 2)
        wait_scatter((NT - 1) % 2)
        plsc.subcore_barrier()

        obase = cid * NP + rbase
        for t in range(RPS // C):
            pltpu.sync_copy(s_sh.at[pl.ds(rbase + t * C, C)],
                            out_hbm.at[pl.ds(obase + t * C, C)])

    return k


def _make_deg_sc():
    @functools.partial(
        pl.kernel,
        mesh=_sc_mesh(),
        out_type=jax.ShapeDtypeStruct((2 * NP, H), jnp.float32),
        scratch_types=[
            pltpu.VMEM_SHARED((NP, H), jnp.float32),
            pltpu.VMEM((CD,), jnp.int32),
            pltpu.VMEM((CD, H), jnp.float32),
        ],
        interpret=_INTERPRET,
    )
    def k(row_hbm, out_hbm, s_sh, idx_r, ones_b):
        cid = lax.axis_index("c")
        sid = lax.axis_index("s")
        wid = sid * NC + cid

        _zero_f32(ones_b, CD)
        rbase = sid * RPS
        for t in range(RPS // CD):
            pltpu.sync_copy(ones_b, s_sh.at[pl.ds(rbase + t * CD, CD)])

        def obody(r, _):
            for k8 in range(H // 16):
                ones_b[r, pl.ds(k8 * 16, 16)] = jnp.ones((16,), jnp.float32)
            return 0
        lax.fori_loop(0, CD, obody, 0)
        plsc.subcore_barrier()

        def chunk_body(t, _):
            ebase = (t * NW + wid) * CD
            pltpu.sync_copy(row_hbm.at[pl.ds(ebase, CD)], idx_r)
            pltpu.sync_copy(ones_b, s_sh.at[idx_r], add=True)
            return 0

        lax.fori_loop(0, NTD, chunk_body, 0)
        plsc.subcore_barrier()

        obase = cid * NP + rbase
        for t in range(RPS // CD):
            pltpu.sync_copy(s_sh.at[pl.ds(rbase + t * CD, CD)],
                            out_hbm.at[pl.ds(obase + t * CD, CD)])

    return k


def _pad8(v):
    return jnp.broadcast_to(v[None, :], (8, H))


def kernel(x, edge_index, edge_attr, batch, params):
    row = edge_index[0]
    col = edge_index[1]
    layers = params["layers"]

    w1a = [p["mW1"][:H] for p in layers]
    w1b = [p["mW1"][H:2 * H] for p in layers]
    wea = jnp.concatenate([p["mW1"][2 * H:] for p in layers] +
                          [jnp.zeros((4, H), jnp.float32)], axis=0)
    mb1s = jnp.concatenate(
        [jnp.stack([p["mb1"] for p in layers]),
         jnp.zeros((5, H), jnp.float32)], axis=0)
    vecs = [jnp.concatenate(
        [jnp.stack([p["mb2"], p["ub1"], p["ub2"], p["ln_g"], p["ln_b"]]),
         jnp.zeros((3, H), jnp.float32)], axis=0) for p in layers]
    uw1a = [p["uW1"][:H] for p in layers]
    uw1b = [p["uW1"][H:] for p in layers]
    x_pad = jnp.pad(x, ((0, NP - N), (0, 0)))
    batch_pad = jnp.pad(batch.astype(jnp.int32), (0, NP - N),
                        constant_values=127)

    h, a_cur, b_cur = _in_proj(x_pad, params["in_W"], _pad8(params["in_b"]),
                               w1a[0], w1b[0])
    ea1, ea2, ea3 = _edge_pre(edge_attr, wea, mb1s)
    eas = [ea1, ea2, ea3]

    sc_edge = _make_edge_sc()
    deg_all = _make_deg_sc()(row)
    d0, d1 = deg_all[:NP], deg_all[NP:]

    for i in range(3):
        s_all = sc_edge(a_cur, b_cur, eas[i], row, col)
        s0, s1 = s_all[:NP], s_all[NP:]
        p = layers[i]
        if i < 2:
            h, a_cur, b_cur = _update(
                i == 0, True, h, s0, s1, d0, d1,
                p["mW2"], uw1a[i], uw1b[i], p["uW2"], vecs[i],
                w1a[i + 1], w1b[i + 1])
        else:
            (h,) = _update(
                False, False, h, s0, s1, d0, d1,
                p["mW2"], uw1a[i], uw1b[i], p["uW2"], vecs[i])

    bid_rep = jnp.broadcast_to(batch_pad[:, None], (NP, H))
    return _pool(h, bid_rep, params["out_W"], _pad8(params["out_b"]))


# speedup vs baseline: 1.8712x; 1.0299x over previous
import functools

import jax
import jax.numpy as jnp
from jax import lax
from jax.experimental import pallas as pl
from jax.experimental.pallas import tpu as pltpu
from jax.experimental.pallas import tpu_sc as plsc

N = 10000
E = 320000
H = 128
ED = 4
NG = 64
NC = 2
NS = 16
NW = NC * NS
C = 80
NT = (E // C) // NW
CD = 80
NTD = (E // CD) // NW
NP = 10240
RPS = NP // NS
RB = 1024
REB = 4000

_INTERPRET = False


def _in_proj_body(x_ref, w_ref, b_ref, wa_ref, wb_ref, h_ref, a_ref, b2_ref):
    h = jnp.dot(x_ref[...], w_ref[...], preferred_element_type=jnp.float32)
    h = jnp.maximum(h + b_ref[0:1, :], 0.0)
    h_ref[...] = h
    a_ref[...] = jnp.dot(h, wa_ref[...], preferred_element_type=jnp.float32)
    b2_ref[...] = jnp.dot(h, wb_ref[...], preferred_element_type=jnp.float32)


def _in_proj(x, in_w, in_b8, w1a, w1b):
    wspec = pl.BlockSpec((H, H), lambda i: (0, 0))
    bspec = pl.BlockSpec((8, H), lambda i: (0, 0))
    nspec = pl.BlockSpec((RB, H), lambda i: (i, 0))
    return pl.pallas_call(
        _in_proj_body,
        grid=(NP // RB,),
        in_specs=[nspec, wspec, bspec, wspec, wspec],
        out_specs=[nspec, nspec, nspec],
        out_shape=[jax.ShapeDtypeStruct((NP, H), jnp.float32)] * 3,
        interpret=_INTERPRET,
    )(x, in_w, in_b8, w1a, w1b)


def _edge_pre_body(attr_ref, w_ref, b_ref, e1_ref, e2_ref, e3_ref):
    a = attr_ref[...]
    outs = [e1_ref, e2_ref, e3_ref]
    for i in range(3):
        w = w_ref[4 * i:4 * (i + 1), :]
        outs[i][...] = (
            jnp.dot(a, w, preferred_element_type=jnp.float32) + b_ref[i:i + 1, :]
        )


def _edge_pre(edge_attr, wea, mb1s):
    espec = pl.BlockSpec((REB, H), lambda i: (i, 0))
    return pl.pallas_call(
        _edge_pre_body,
        grid=(E // REB,),
        in_specs=[
            pl.BlockSpec((REB, ED), lambda i: (i, 0)),
            pl.BlockSpec((16, H), lambda i: (0, 0)),
            pl.BlockSpec((8, H), lambda i: (0, 0)),
        ],
        out_specs=[espec, espec, espec],
        out_shape=[jax.ShapeDtypeStruct((E, H), jnp.float32)] * 3,
        interpret=_INTERPRET,
    )(edge_attr, wea, mb1s)


def _make_update_body(first, has_next):
    def body(h_ref, s0_ref, s1_ref, d0_ref, d1_ref, mw2_ref, uw1a_ref,
             uw1b_ref, uw2_ref, vec_ref, *rest):
        h = h_ref[...]
        s = s0_ref[...] + s1_ref[...]
        deg = (d0_ref[...] + d1_ref[...]).astype(jnp.float32)[:, 0:1]
        aggr = jnp.dot(s, mw2_ref[...], preferred_element_type=jnp.float32)
        aggr = aggr + deg * vec_ref[0:1, :]
        t = jnp.dot(h, uw1a_ref[...], preferred_element_type=jnp.float32)
        t = t + jnp.dot(aggr, uw1b_ref[...], preferred_element_type=jnp.float32)
        t = jnp.maximum(t + vec_ref[1:2, :], 0.0)
        u = jnp.dot(t, uw2_ref[...], preferred_element_type=jnp.float32)
        u = u + vec_ref[2:3, :]
        mu = jnp.mean(u, axis=1, keepdims=True)
        var = jnp.mean((u - mu) ** 2, axis=1, keepdims=True)
        un = (u - mu) * lax.rsqrt(var + 1e-5) * vec_ref[3:4, :] + vec_ref[4:5, :]
        hn = jnp.maximum(un, 0.0)
        if not first:
            hn = hn + h
        if has_next:
            wa_ref, wb_ref, hn_ref, a_ref, b_ref = rest
            hn_ref[...] = hn
            a_ref[...] = jnp.dot(hn, wa_ref[...], preferred_element_type=jnp.float32)
            b_ref[...] = jnp.dot(hn, wb_ref[...], preferred_element_type=jnp.float32)
        else:
            bid_ref, ow_ref, ob_ref, o_ref, acc_s, acc_c = rest
            i = pl.program_id(0)

            @pl.when(i == 0)
            def _():
                acc_s[...] = jnp.zeros_like(acc_s)
                acc_c[...] = jnp.zeros_like(acc_c)

            lanes = lax.broadcasted_iota(jnp.int32, (RB, H), 1)
            onehot = (bid_ref[...] == lanes).astype(jnp.float32)
            dn = (((0,), (0,)), ((), ()))
            acc_s[...] += lax.dot_general(onehot, hn, dn,
                                          preferred_element_type=jnp.float32)
            acc_c[...] += lax.dot_general(onehot, jnp.ones_like(hn), dn,
                                          preferred_element_type=jnp.float32)

            @pl.when(i == pl.num_programs(0) - 1)
            def _():
                rep = acc_s[...] / jnp.maximum(acc_c[...], 1.0)
                o_ref[...] = (
                    jnp.dot(rep[:NG, :], ow_ref[...],
                            preferred_element_type=jnp.float32)
                    + ob_ref[0:1, :]
                )
    return body


def _update(first, has_next, h, s0, s1, d0, d1, mw2, uw1a, uw1b, uw2, vec,
            wa_next=None, wb_next=None, bid_rep=None, out_w=None,
            out_b8=None):
    nspec = pl.BlockSpec((RB, H), lambda i: (i, 0))
    wspec = pl.BlockSpec((H, H), lambda i: (0, 0))
    vspec = pl.BlockSpec((8, H), lambda i: (0, 0))
    in_specs = [nspec, nspec, nspec, nspec, nspec, wspec, wspec, wspec,
                wspec, vspec]
    args = [h, s0, s1, d0, d1, mw2, uw1a, uw1b, uw2, vec]
    scratch = []
    if has_next:
        in_specs += [wspec, wspec]
        args += [wa_next, wb_next]
        out_specs = [nspec, nspec, nspec]
        out_shape = [jax.ShapeDtypeStruct((NP, H), jnp.float32)] * 3
    else:
        in_specs += [nspec, wspec, vspec]
        args += [bid_rep, out_w, out_b8]
        out_specs = pl.BlockSpec((NG, H), lambda i: (0, 0))
        out_shape = jax.ShapeDtypeStruct((NG, H), jnp.float32)
        scratch = [pltpu.VMEM((H, H), jnp.float32),
                   pltpu.VMEM((H, H), jnp.float32)]
    return pl.pallas_call(
        _make_update_body(first, has_next),
        grid=(NP // RB,),
        in_specs=in_specs,
        out_specs=out_specs,
        out_shape=out_shape,
        scratch_shapes=scratch,
        interpret=_INTERPRET,
    )(*args)


def _sc_mesh():
    return plsc.VectorSubcoreMesh(
        core_axis_name="c", subcore_axis_name="s",
        num_cores=NC, num_subcores=NS)


def _zero_f32(buf, rows):
    def zbody(r, _):
        for k8 in range(H // 16):
            buf[r, pl.ds(k8 * 16, 16)] = jnp.zeros((16,), jnp.float32)
        return 0
    lax.fori_loop(0, rows, zbody, 0)


def _make_edge_sc():
    @functools.partial(
        pl.kernel,
        mesh=_sc_mesh(),
        out_type=jax.ShapeDtypeStruct((2 * NP, H), jnp.float32),
        scratch_types=[
            pltpu.VMEM_SHARED((NP, H), jnp.float32),
            [pltpu.VMEM((C,), jnp.int32)] * 2,
            pltpu.VMEM((C,), jnp.int32),
            pltpu.VMEM((C, H), jnp.float32),
            pltpu.VMEM((C, H), jnp.float32),
            pltpu.VMEM((C, H), jnp.float32),
            pltpu.VMEM((C, H), jnp.float32),
            pltpu.SemaphoreType.DMA,
            pltpu.SemaphoreType.DMA,
            pltpu.SemaphoreType.DMA,
        ],
        interpret=_INTERPRET,
    )
    def k(a_hbm, b_hbm, e_hbm, row_hbm, col_hbm, out_hbm,
          s_sh, idx_r, idx_c, buf_a, buf_b, buf_e, buf_o,
          sem_a, sem_b, sem_s):
        cid = lax.axis_index("c")
        sid = lax.axis_index("s")
        wid = sid * NC + cid

        _zero_f32(buf_o, C)
        rbase = sid * RPS
        for t in range(RPS // C):
            pltpu.sync_copy(buf_o, s_sh.at[pl.ds(rbase + t * C, C)])
        plsc.subcore_barrier()

        def wait_scatter(par):
            pltpu.make_async_copy(buf_o, s_sh.at[idx_r[par]], sem_s).wait()

        def do_chunk(t, par):
            ebase = (t * NW + wid) * C
            pltpu.sync_copy(row_hbm.at[pl.ds(ebase, C)], idx_r[par])
            pltpu.sync_copy(col_hbm.at[pl.ds(ebase, C)], idx_c)
            cp_a = pltpu.async_copy(a_hbm.at[idx_r[par]], buf_a, sem_a)
            cp_b = pltpu.async_copy(b_hbm.at[idx_c], buf_b, sem_b)
            pltpu.sync_copy(e_hbm.at[pl.ds(ebase, C)], buf_e)
            cp_a.wait()
            cp_b.wait()

            @pl.when(t > 0)
            def _():
                wait_scatter(1 - par)

            def comp(r, _):
                for k8 in range(H // 16):
                    sl = pl.ds(k8 * 16, 16)
                    v = buf_a[r, sl] + buf_b[r, sl] + buf_e[r, sl]
                    buf_o[r, sl] = jnp.maximum(v, 0.0)
                return 0

            lax.fori_loop(0, C, comp, 0)
            pltpu.async_copy(buf_o, s_sh.at[idx_r[par]], sem_s, add=True)

        def pair_body(j, _):
            for par in range(2):
                t = j * 2 + par
                do_chunk(t, par)
            return 0

        lax.fori_loop(0, NT // 2, pair_body, 0)
        do_chunk(NT - 1, (NT - 1) % 2)
        wait_scatter((NT - 1) % 2)
        plsc.subcore_barrier()

        obase = cid * NP + rbase
        for t in range(RPS // C):
            pltpu.sync_copy(s_sh.at[pl.ds(rbase + t * C, C)],
                            out_hbm.at[pl.ds(obase + t * C, C)])

    return k


def _make_deg_sc():
    @functools.partial(
        pl.kernel,
        mesh=_sc_mesh(),
        out_type=jax.ShapeDtypeStruct((2 * NP, H), jnp.float32),
        scratch_types=[
            pltpu.VMEM_SHARED((NP, H), jnp.float32),
            [pltpu.VMEM((CD,), jnp.int32)] * 2,
            pltpu.VMEM((CD, H), jnp.float32),
            pltpu.SemaphoreType.DMA,
        ],
        interpret=_INTERPRET,
    )
    def k(row_hbm, out_hbm, s_sh, idx_r, ones_b, sem_s):
        cid = lax.axis_index("c")
        sid = lax.axis_index("s")
        wid = sid * NC + cid

        _zero_f32(ones_b, CD)
        rbase = sid * RPS
        for t in range(RPS // CD):
            pltpu.sync_copy(ones_b, s_sh.at[pl.ds(rbase + t * CD, CD)])

        def obody(r, _):
            for k8 in range(H // 16):
                ones_b[r, pl.ds(k8 * 16, 16)] = jnp.ones((16,), jnp.float32)
            return 0
        lax.fori_loop(0, CD, obody, 0)
        plsc.subcore_barrier()

        def wait_scatter():
            pltpu.make_async_copy(ones_b, s_sh.at[idx_r[0]], sem_s).wait()

        def do_chunk(t, par):
            ebase = (t * NW + wid) * CD
            pltpu.sync_copy(row_hbm.at[pl.ds(ebase, CD)], idx_r[par])

            @pl.when(t > 1)
            def _():
                wait_scatter()
            pltpu.async_copy(ones_b, s_sh.at[idx_r[par]], sem_s, add=True)

        def pair_body(j, _):
            for par in range(2):
                do_chunk(j * 2 + par, par)
            return 0

        lax.fori_loop(0, NTD // 2, pair_body, 0)
        do_chunk(NTD - 1, (NTD - 1) % 2)
        wait_scatter()
        wait_scatter()
        plsc.subcore_barrier()

        obase = cid * NP + rbase
        for t in range(RPS // CD):
            pltpu.sync_copy(s_sh.at[pl.ds(rbase + t * CD, CD)],
                            out_hbm.at[pl.ds(obase + t * CD, CD)])

    return k


def _pad8(v):
    return jnp.broadcast_to(v[None, :], (8, H))


def kernel(x, edge_index, edge_attr, batch, params):
    row = edge_index[0]
    col = edge_index[1]
    layers = params["layers"]

    w1a = [p["mW1"][:H] for p in layers]
    w1b = [p["mW1"][H:2 * H] for p in layers]
    wea = jnp.concatenate([p["mW1"][2 * H:] for p in layers] +
                          [jnp.zeros((4, H), jnp.float32)], axis=0)
    mb1s = jnp.concatenate(
        [jnp.stack([p["mb1"] for p in layers]),
         jnp.zeros((5, H), jnp.float32)], axis=0)
    vecs = [jnp.concatenate(
        [jnp.stack([p["mb2"], p["ub1"], p["ub2"], p["ln_g"], p["ln_b"]]),
         jnp.zeros((3, H), jnp.float32)], axis=0) for p in layers]
    uw1a = [p["uW1"][:H] for p in layers]
    uw1b = [p["uW1"][H:] for p in layers]
    x_pad = jnp.pad(x, ((0, NP - N), (0, 0)))
    batch_pad = jnp.pad(batch.astype(jnp.int32), (0, NP - N),
                        constant_values=127)

    h, a_cur, b_cur = _in_proj(x_pad, params["in_W"], _pad8(params["in_b"]),
                               w1a[0], w1b[0])
    ea1, ea2, ea3 = _edge_pre(edge_attr, wea, mb1s)
    eas = [ea1, ea2, ea3]

    sc_edge = _make_edge_sc()
    deg_all = _make_deg_sc()(row)
    d0, d1 = deg_all[:NP], deg_all[NP:]

    for i in range(3):
        s_all = sc_edge(a_cur, b_cur, eas[i], row, col)
        s0, s1 = s_all[:NP], s_all[NP:]
        p = layers[i]
        if i < 2:
            h, a_cur, b_cur = _update(
                i == 0, True, h, s0, s1, d0, d1,
                p["mW2"], uw1a[i], uw1b[i], p["uW2"], vecs[i],
                w1a[i + 1], w1b[i + 1])
        else:
            bid_rep = jnp.broadcast_to(batch_pad[:, None], (NP, H))
            out = _update(
                False, False, h, s0, s1, d0, d1,
                p["mW2"], uw1a[i], uw1b[i], p["uW2"], vecs[i],
                bid_rep=bid_rep, out_w=params["out_W"],
                out_b8=_pad8(params["out_b"]))

    return out


# speedup vs baseline: 2.2515x; 1.2032x over previous
import functools

import jax
import jax.numpy as jnp
from jax import lax
from jax.experimental import pallas as pl
from jax.experimental.pallas import tpu as pltpu
from jax.experimental.pallas import tpu_sc as plsc

N = 10000
E = 320000
H = 128
ED = 4
NG = 64
NC = 2
NS = 16
NW = NC * NS
C = 80
NT = (E // C) // NW
CD = 80
NTD = (E // CD) // NW
NP = 10240
RPS = NP // NS
RB = 1024
REB = 4000

_INTERPRET = False


def _in_proj_body(x_ref, w_ref, b_ref, wa_ref, wb_ref, h_ref, a_ref, b2_ref):
    h = jnp.dot(x_ref[...], w_ref[...], preferred_element_type=jnp.float32)
    h = jnp.maximum(h + b_ref[0:1, :], 0.0)
    h_ref[...] = h
    a_ref[...] = jnp.dot(h, wa_ref[...], preferred_element_type=jnp.float32)
    b2_ref[...] = jnp.dot(h, wb_ref[...], preferred_element_type=jnp.float32)


def _in_proj(x, in_w, in_b8, w1a, w1b):
    wspec = pl.BlockSpec((H, H), lambda i: (0, 0))
    bspec = pl.BlockSpec((8, H), lambda i: (0, 0))
    nspec = pl.BlockSpec((RB, H), lambda i: (i, 0))
    return pl.pallas_call(
        _in_proj_body,
        grid=(NP // RB,),
        in_specs=[nspec, wspec, bspec, wspec, wspec],
        out_specs=[nspec, nspec, nspec],
        out_shape=[jax.ShapeDtypeStruct((NP, H), jnp.float32)] * 3,
        interpret=_INTERPRET,
    )(x, in_w, in_b8, w1a, w1b)


def _edge_pre_body(attr_ref, w_ref, b_ref, e1_ref, e2_ref, e3_ref):
    a = attr_ref[...]
    outs = [e1_ref, e2_ref, e3_ref]
    for i in range(3):
        w = w_ref[4 * i:4 * (i + 1), :]
        outs[i][...] = (
            jnp.dot(a, w, preferred_element_type=jnp.float32) + b_ref[i:i + 1, :]
        )


def _edge_pre(edge_attr, wea, mb1s):
    espec = pl.BlockSpec((REB, H), lambda i: (i, 0))
    return pl.pallas_call(
        _edge_pre_body,
        grid=(E // REB,),
        in_specs=[
            pl.BlockSpec((REB, ED), lambda i: (i, 0)),
            pl.BlockSpec((16, H), lambda i: (0, 0)),
            pl.BlockSpec((8, H), lambda i: (0, 0)),
        ],
        out_specs=[espec, espec, espec],
        out_shape=[jax.ShapeDtypeStruct((E, H), jnp.float32)] * 3,
        interpret=_INTERPRET,
    )(edge_attr, wea, mb1s)


def _make_update_body(first, has_next):
    def body(h_ref, s0_ref, s1_ref, d0_ref, d1_ref, mw2_ref, uw1a_ref,
             uw1b_ref, uw2_ref, vec_ref, *rest):
        h = h_ref[...]
        s = s0_ref[...] + s1_ref[...]
        deg = (d0_ref[...] + d1_ref[...]).astype(jnp.float32)[:, 0:1]
        aggr = jnp.dot(s, mw2_ref[...], preferred_element_type=jnp.float32)
        aggr = aggr + deg * vec_ref[0:1, :]
        t = jnp.dot(h, uw1a_ref[...], preferred_element_type=jnp.float32)
        t = t + jnp.dot(aggr, uw1b_ref[...], preferred_element_type=jnp.float32)
        t = jnp.maximum(t + vec_ref[1:2, :], 0.0)
        u = jnp.dot(t, uw2_ref[...], preferred_element_type=jnp.float32)
        u = u + vec_ref[2:3, :]
        mu = jnp.mean(u, axis=1, keepdims=True)
        var = jnp.mean((u - mu) ** 2, axis=1, keepdims=True)
        un = (u - mu) * lax.rsqrt(var + 1e-5) * vec_ref[3:4, :] + vec_ref[4:5, :]
        hn = jnp.maximum(un, 0.0)
        if not first:
            hn = hn + h
        if has_next:
            wa_ref, wb_ref, hn_ref, a_ref, b_ref = rest
            hn_ref[...] = hn
            a_ref[...] = jnp.dot(hn, wa_ref[...], preferred_element_type=jnp.float32)
            b_ref[...] = jnp.dot(hn, wb_ref[...], preferred_element_type=jnp.float32)
        else:
            bid_ref, ow_ref, ob_ref, o_ref, acc_s, acc_c = rest
            i = pl.program_id(0)

            @pl.when(i == 0)
            def _():
                acc_s[...] = jnp.zeros_like(acc_s)
                acc_c[...] = jnp.zeros_like(acc_c)

            lanes = lax.broadcasted_iota(jnp.int32, (RB, H), 1)
            onehot = (bid_ref[...] == lanes).astype(jnp.float32)
            dn = (((0,), (0,)), ((), ()))
            acc_s[...] += lax.dot_general(onehot, hn, dn,
                                          preferred_element_type=jnp.float32)
            acc_c[...] += lax.dot_general(onehot, jnp.ones_like(hn), dn,
                                          preferred_element_type=jnp.float32)

            @pl.when(i == pl.num_programs(0) - 1)
            def _():
                rep = acc_s[...] / jnp.maximum(acc_c[...], 1.0)
                o_ref[...] = (
                    jnp.dot(rep[:NG, :], ow_ref[...],
                            preferred_element_type=jnp.float32)
                    + ob_ref[0:1, :]
                )
    return body


def _update(first, has_next, h, s0, s1, d0, d1, mw2, uw1a, uw1b, uw2, vec,
            wa_next=None, wb_next=None, bid_rep=None, out_w=None,
            out_b8=None):
    nspec = pl.BlockSpec((RB, H), lambda i: (i, 0))
    wspec = pl.BlockSpec((H, H), lambda i: (0, 0))
    vspec = pl.BlockSpec((8, H), lambda i: (0, 0))
    in_specs = [nspec, nspec, nspec, nspec, nspec, wspec, wspec, wspec,
                wspec, vspec]
    args = [h, s0, s1, d0, d1, mw2, uw1a, uw1b, uw2, vec]
    scratch = []
    if has_next:
        in_specs += [wspec, wspec]
        args += [wa_next, wb_next]
        out_specs = [nspec, nspec, nspec]
        out_shape = [jax.ShapeDtypeStruct((NP, H), jnp.float32)] * 3
    else:
        in_specs += [nspec, wspec, vspec]
        args += [bid_rep, out_w, out_b8]
        out_specs = pl.BlockSpec((NG, H), lambda i: (0, 0))
        out_shape = jax.ShapeDtypeStruct((NG, H), jnp.float32)
        scratch = [pltpu.VMEM((H, H), jnp.float32),
                   pltpu.VMEM((H, H), jnp.float32)]
    return pl.pallas_call(
        _make_update_body(first, has_next),
        grid=(NP // RB,),
        in_specs=in_specs,
        out_specs=out_specs,
        out_shape=out_shape,
        scratch_shapes=scratch,
        interpret=_INTERPRET,
    )(*args)


def _sc_mesh():
    return plsc.VectorSubcoreMesh(
        core_axis_name="c", subcore_axis_name="s",
        num_cores=NC, num_subcores=NS)


def _zero_f32(buf, rows):
    def zbody(r, _):
        for k8 in range(H // 16):
            buf[r, pl.ds(k8 * 16, 16)] = jnp.zeros((16,), jnp.float32)
        return 0
    lax.fori_loop(0, rows, zbody, 0)


def _make_edge_sc():
    @functools.partial(
        pl.kernel,
        mesh=_sc_mesh(),
        out_type=jax.ShapeDtypeStruct((2 * NP, H), jnp.float32),
        scratch_types=[
            pltpu.VMEM_SHARED((NP, H), jnp.float32),
            [pltpu.VMEM((C,), jnp.int32)] * 2,
            [pltpu.VMEM((C,), jnp.int32)] * 2,
            pltpu.VMEM((C, H), jnp.float32),
            pltpu.VMEM((C, H), jnp.float32),
            pltpu.VMEM((C, H), jnp.float32),
            pltpu.VMEM((C, H), jnp.float32),
            pltpu.SemaphoreType.DMA,
            pltpu.SemaphoreType.DMA,
            pltpu.SemaphoreType.DMA,
            pltpu.SemaphoreType.DMA,
            pltpu.SemaphoreType.DMA,
        ],
        interpret=_INTERPRET,
    )
    def k(a_hbm, b_hbm, e_hbm, row_hbm, col_hbm, out_hbm,
          s_sh, idx_r, idx_c, buf_a, buf_b, buf_e, buf_o,
          sem_a, sem_b, sem_s, sem_ir, sem_ic):
        cid = lax.axis_index("c")
        sid = lax.axis_index("s")
        wid = sid * NC + cid

        _zero_f32(buf_o, C)
        rbase = sid * RPS
        for t in range(RPS // C):
            pltpu.sync_copy(buf_o, s_sh.at[pl.ds(rbase + t * C, C)])
        plsc.subcore_barrier()

        def wait_scatter(par):
            pltpu.make_async_copy(buf_o, s_sh.at[idx_r[par]], sem_s).wait()

        def load_idx(t, par):
            ebase = (t * NW + wid) * C
            pltpu.async_copy(row_hbm.at[pl.ds(ebase, C)], idx_r[par], sem_ir)
            pltpu.async_copy(col_hbm.at[pl.ds(ebase, C)], idx_c[par], sem_ic)

        def wait_idx(par):
            pltpu.make_async_copy(row_hbm.at[pl.ds(0, C)], idx_r[par],
                                  sem_ir).wait()
            pltpu.make_async_copy(col_hbm.at[pl.ds(0, C)], idx_c[par],
                                  sem_ic).wait()

        def do_chunk(t, par):
            wait_idx(par)
            cp_a = pltpu.async_copy(a_hbm.at[idx_r[par]], buf_a, sem_a)
            cp_b = pltpu.async_copy(b_hbm.at[idx_c[par]], buf_b, sem_b)
            ebase = (t * NW + wid) * C
            pltpu.sync_copy(e_hbm.at[pl.ds(ebase, C)], buf_e)

            @pl.when(t > 0)
            def _():
                wait_scatter(1 - par)

            @pl.when(t < NT - 1)
            def _():
                load_idx(t + 1, 1 - par)

            cp_a.wait()
            cp_b.wait()

            def comp(r, _):
                for k8 in range(H // 16):
                    sl = pl.ds(k8 * 16, 16)
                    v = buf_a[r, sl] + buf_b[r, sl] + buf_e[r, sl]
                    buf_o[r, sl] = jnp.maximum(v, 0.0)
                return 0

            lax.fori_loop(0, C, comp, 0)
            pltpu.async_copy(buf_o, s_sh.at[idx_r[par]], sem_s, add=True)

        load_idx(0, 0)

        def pair_body(j, _):
            for par in range(2):
                t = j * 2 + par
                do_chunk(t, par)
            return 0

        lax.fori_loop(0, NT // 2, pair_body, 0)
        do_chunk(NT - 1, (NT - 1) % 2)
        wait_scatter((NT - 1) % 2)
        plsc.subcore_barrier()

        obase = cid * NP + rbase
        for t in range(RPS // C):
            pltpu.sync_copy(s_sh.at[pl.ds(rbase + t * C, C)],
                            out_hbm.at[pl.ds(obase + t * C, C)])

    return k


def _make_deg_sc():
    @functools.partial(
        pl.kernel,
        mesh=_sc_mesh(),
        out_type=jax.ShapeDtypeStruct((2 * NP, H), jnp.float32),
        scratch_types=[
            pltpu.VMEM_SHARED((NP, H), jnp.float32),
            [pltpu.VMEM((CD,), jnp.int32)] * 2,
            pltpu.VMEM((CD, H), jnp.float32),
            pltpu.SemaphoreType.DMA,
        ],
        interpret=_INTERPRET,
    )
    def k(row_hbm, out_hbm, s_sh, idx_r, ones_b, sem_s):
        cid = lax.axis_index("c")
        sid = lax.axis_index("s")
        wid = sid * NC + cid

        _zero_f32(ones_b, CD)
        rbase = sid * RPS
        for t in range(RPS // CD):
            pltpu.sync_copy(ones_b, s_sh.at[pl.ds(rbase + t * CD, CD)])

        def obody(r, _):
            for k8 in range(H // 16):
                ones_b[r, pl.ds(k8 * 16, 16)] = jnp.ones((16,), jnp.float32)
            return 0
        lax.fori_loop(0, CD, obody, 0)
        plsc.subcore_barrier()

        def wait_scatter():
            pltpu.make_async_copy(ones_b, s_sh.at[idx_r[0]], sem_s).wait()

        def do_chunk(t, par):
            ebase = (t * NW + wid) * CD
            pltpu.sync_copy(row_hbm.at[pl.ds(ebase, CD)], idx_r[par])

            @pl.when(t > 1)
            def _():
                wait_scatter()
            pltpu.async_copy(ones_b, s_sh.at[idx_r[par]], sem_s, add=True)

        def pair_body(j, _):
            for par in range(2):
                do_chunk(j * 2 + par, par)
            return 0

        lax.fori_loop(0, NTD // 2, pair_body, 0)
        do_chunk(NTD - 1, (NTD - 1) % 2)
        wait_scatter()
        wait_scatter()
        plsc.subcore_barrier()

        obase = cid * NP + rbase
        for t in range(RPS // CD):
            pltpu.sync_copy(s_sh.at[pl.ds(rbase + t * CD, CD)],
                            out_hbm.at[pl.ds(obase + t * CD, CD)])

    return k


def _pad8(v):
    return jnp.broadcast_to(v[None, :], (8, H))


def kernel(x, edge_index, edge_attr, batch, params):
    row = edge_index[0]
    col = edge_index[1]
    layers = params["layers"]

    w1a = [p["mW1"][:H] for p in layers]
    w1b = [p["mW1"][H:2 * H] for p in layers]
    wea = jnp.concatenate([p["mW1"][2 * H:] for p in layers] +
                          [jnp.zeros((4, H), jnp.float32)], axis=0)
    mb1s = jnp.concatenate(
        [jnp.stack([p["mb1"] for p in layers]),
         jnp.zeros((5, H), jnp.float32)], axis=0)
    vecs = [jnp.concatenate(
        [jnp.stack([p["mb2"], p["ub1"], p["ub2"], p["ln_g"], p["ln_b"]]),
         jnp.zeros((3, H), jnp.float32)], axis=0) for p in layers]
    uw1a = [p["uW1"][:H] for p in layers]
    uw1b = [p["uW1"][H:] for p in layers]
    x_pad = jnp.pad(x, ((0, NP - N), (0, 0)))
    batch_pad = jnp.pad(batch.astype(jnp.int32), (0, NP - N),
                        constant_values=127)

    h, a_cur, b_cur = _in_proj(x_pad, params["in_W"], _pad8(params["in_b"]),
                               w1a[0], w1b[0])
    ea1, ea2, ea3 = _edge_pre(edge_attr, wea, mb1s)
    eas = [ea1, ea2, ea3]

    sc_edge = _make_edge_sc()
    deg_all = _make_deg_sc()(row)
    d0, d1 = deg_all[:NP], deg_all[NP:]

    for i in range(3):
        s_all = sc_edge(a_cur, b_cur, eas[i], row, col)
        s0, s1 = s_all[:NP], s_all[NP:]
        p = layers[i]
        if i < 2:
            h, a_cur, b_cur = _update(
                i == 0, True, h, s0, s1, d0, d1,
                p["mW2"], uw1a[i], uw1b[i], p["uW2"], vecs[i],
                w1a[i + 1], w1b[i + 1])
        else:
            bid_rep = jnp.broadcast_to(batch_pad[:, None], (NP, H))
            out = _update(
                False, False, h, s0, s1, d0, d1,
                p["mW2"], uw1a[i], uw1b[i], p["uW2"], vecs[i],
                bid_rep=bid_rep, out_w=params["out_W"],
                out_b8=_pad8(params["out_b"]))

    return out


# speedup vs baseline: 2.5157x; 1.1174x over previous
import functools

import jax
import jax.numpy as jnp
from jax import lax
from jax.experimental import pallas as pl
from jax.experimental.pallas import tpu as pltpu
from jax.experimental.pallas import tpu_sc as plsc

N = 10000
E = 320000
H = 128
ED = 4
NG = 64
NC = 2
NS = 16
NW = NC * NS
C = 40
NT = (E // C) // NW
CD = 80
NTD = (E // CD) // NW
NP = 10240
RPS = NP // NS
RB = 1024
REB = 4000

_INTERPRET = False


def _in_proj_body(x_ref, w_ref, b_ref, wa_ref, wb_ref, h_ref, a_ref, b2_ref):
    h = jnp.dot(x_ref[...], w_ref[...], preferred_element_type=jnp.float32)
    h = jnp.maximum(h + b_ref[0:1, :], 0.0)
    h_ref[...] = h
    a_ref[...] = jnp.dot(h, wa_ref[...], preferred_element_type=jnp.float32)
    b2_ref[...] = jnp.dot(h, wb_ref[...], preferred_element_type=jnp.float32)


def _in_proj(x, in_w, in_b8, w1a, w1b):
    wspec = pl.BlockSpec((H, H), lambda i: (0, 0))
    bspec = pl.BlockSpec((8, H), lambda i: (0, 0))
    nspec = pl.BlockSpec((RB, H), lambda i: (i, 0))
    return pl.pallas_call(
        _in_proj_body,
        grid=(NP // RB,),
        in_specs=[nspec, wspec, bspec, wspec, wspec],
        out_specs=[nspec, nspec, nspec],
        out_shape=[jax.ShapeDtypeStruct((NP, H), jnp.float32)] * 3,
        interpret=_INTERPRET,
    )(x, in_w, in_b8, w1a, w1b)


def _edge_pre_body(attr_ref, w_ref, b_ref, e1_ref, e2_ref, e3_ref):
    a = attr_ref[...]
    outs = [e1_ref, e2_ref, e3_ref]
    for i in range(3):
        w = w_ref[4 * i:4 * (i + 1), :]
        outs[i][...] = (
            jnp.dot(a, w, preferred_element_type=jnp.float32) + b_ref[i:i + 1, :]
        )


def _edge_pre(edge_attr, wea, mb1s):
    espec = pl.BlockSpec((REB, H), lambda i: (i, 0))
    return pl.pallas_call(
        _edge_pre_body,
        grid=(E // REB,),
        in_specs=[
            pl.BlockSpec((REB, ED), lambda i: (i, 0)),
            pl.BlockSpec((16, H), lambda i: (0, 0)),
            pl.BlockSpec((8, H), lambda i: (0, 0)),
        ],
        out_specs=[espec, espec, espec],
        out_shape=[jax.ShapeDtypeStruct((E, H), jnp.float32)] * 3,
        interpret=_INTERPRET,
    )(edge_attr, wea, mb1s)


def _make_update_body(first, has_next):
    def body(h_ref, s0_ref, s1_ref, d0_ref, d1_ref, mw2_ref, uw1a_ref,
             uw1b_ref, uw2_ref, vec_ref, *rest):
        h = h_ref[...]
        s = s0_ref[...] + s1_ref[...]
        deg = (d0_ref[...] + d1_ref[...]).astype(jnp.float32)[:, 0:1]
        aggr = jnp.dot(s, mw2_ref[...], preferred_element_type=jnp.float32)
        aggr = aggr + deg * vec_ref[0:1, :]
        t = jnp.dot(h, uw1a_ref[...], preferred_element_type=jnp.float32)
        t = t + jnp.dot(aggr, uw1b_ref[...], preferred_element_type=jnp.float32)
        t = jnp.maximum(t + vec_ref[1:2, :], 0.0)
        u = jnp.dot(t, uw2_ref[...], preferred_element_type=jnp.float32)
        u = u + vec_ref[2:3, :]
        mu = jnp.mean(u, axis=1, keepdims=True)
        var = jnp.mean((u - mu) ** 2, axis=1, keepdims=True)
        un = (u - mu) * lax.rsqrt(var + 1e-5) * vec_ref[3:4, :] + vec_ref[4:5, :]
        hn = jnp.maximum(un, 0.0)
        if not first:
            hn = hn + h
        if has_next:
            wa_ref, wb_ref, hn_ref, a_ref, b_ref = rest
            hn_ref[...] = hn
            a_ref[...] = jnp.dot(hn, wa_ref[...], preferred_element_type=jnp.float32)
            b_ref[...] = jnp.dot(hn, wb_ref[...], preferred_element_type=jnp.float32)
        else:
            bid_ref, ow_ref, ob_ref, o_ref, acc_s, acc_c = rest
            i = pl.program_id(0)

            @pl.when(i == 0)
            def _():
                acc_s[...] = jnp.zeros_like(acc_s)
                acc_c[...] = jnp.zeros_like(acc_c)

            lanes = lax.broadcasted_iota(jnp.int32, (RB, H), 1)
            onehot = (bid_ref[...] == lanes).astype(jnp.float32)
            dn = (((0,), (0,)), ((), ()))
            acc_s[...] += lax.dot_general(onehot, hn, dn,
                                          preferred_element_type=jnp.float32)
            acc_c[...] += lax.dot_general(onehot, jnp.ones_like(hn), dn,
                                          preferred_element_type=jnp.float32)

            @pl.when(i == pl.num_programs(0) - 1)
            def _():
                rep = acc_s[...] / jnp.maximum(acc_c[...], 1.0)
                o_ref[...] = (
                    jnp.dot(rep[:NG, :], ow_ref[...],
                            preferred_element_type=jnp.float32)
                    + ob_ref[0:1, :]
                )
    return body


def _update(first, has_next, h, s0, s1, d0, d1, mw2, uw1a, uw1b, uw2, vec,
            wa_next=None, wb_next=None, bid_rep=None, out_w=None,
            out_b8=None):
    nspec = pl.BlockSpec((RB, H), lambda i: (i, 0))
    wspec = pl.BlockSpec((H, H), lambda i: (0, 0))
    vspec = pl.BlockSpec((8, H), lambda i: (0, 0))
    in_specs = [nspec, nspec, nspec, nspec, nspec, wspec, wspec, wspec,
                wspec, vspec]
    args = [h, s0, s1, d0, d1, mw2, uw1a, uw1b, uw2, vec]
    scratch = []
    if has_next:
        in_specs += [wspec, wspec]
        args += [wa_next, wb_next]
        out_specs = [nspec, nspec, nspec]
        out_shape = [jax.ShapeDtypeStruct((NP, H), jnp.float32)] * 3
    else:
        in_specs += [nspec, wspec, vspec]
        args += [bid_rep, out_w, out_b8]
        out_specs = pl.BlockSpec((NG, H), lambda i: (0, 0))
        out_shape = jax.ShapeDtypeStruct((NG, H), jnp.float32)
        scratch = [pltpu.VMEM((H, H), jnp.float32),
                   pltpu.VMEM((H, H), jnp.float32)]
    return pl.pallas_call(
        _make_update_body(first, has_next),
        grid=(NP // RB,),
        in_specs=in_specs,
        out_specs=out_specs,
        out_shape=out_shape,
        scratch_shapes=scratch,
        interpret=_INTERPRET,
    )(*args)


def _sc_mesh():
    return plsc.VectorSubcoreMesh(
        core_axis_name="c", subcore_axis_name="s",
        num_cores=NC, num_subcores=NS)


def _zero_f32(buf, rows):
    def zbody(r, _):
        for k8 in range(H // 16):
            buf[r, pl.ds(k8 * 16, 16)] = jnp.zeros((16,), jnp.float32)
        return 0
    lax.fori_loop(0, rows, zbody, 0)


def _make_edge_sc():
    @functools.partial(
        pl.kernel,
        mesh=_sc_mesh(),
        out_type=jax.ShapeDtypeStruct((2 * NP, H), jnp.float32),
        scratch_types=[
            pltpu.VMEM_SHARED((NP, H), jnp.float32),
            [pltpu.VMEM((C,), jnp.int32)] * 4,
            [pltpu.VMEM((C,), jnp.int32)] * 4,
            [pltpu.VMEM((C, H), jnp.float32)] * 2,
            [pltpu.VMEM((C, H), jnp.float32)] * 2,
            [pltpu.VMEM((C, H), jnp.float32)] * 2,
            pltpu.VMEM((C, H), jnp.float32),
            pltpu.SemaphoreType.DMA,
            pltpu.SemaphoreType.DMA,
            pltpu.SemaphoreType.DMA,
            pltpu.SemaphoreType.DMA,
            [pltpu.SemaphoreType.DMA] * 2,
            [pltpu.SemaphoreType.DMA] * 2,
        ],
        interpret=_INTERPRET,
    )
    def k(a_hbm, b_hbm, e_hbm, row_hbm, col_hbm, out_hbm,
          s_sh, idx_r, idx_c, buf_a, buf_b, buf_e, buf_o,
          sem_a, sem_b, sem_e, sem_s, sem_ir, sem_ic):
        cid = lax.axis_index("c")
        sid = lax.axis_index("s")
        wid = sid * NC + cid

        _zero_f32(buf_o, C)
        rbase = sid * RPS
        for t in range(RPS // 40):
            pltpu.sync_copy(buf_o.at[pl.ds(0, 40)],
                            s_sh.at[pl.ds(rbase + t * 40, 40)])
        plsc.subcore_barrier()

        def wait_scatter():
            pltpu.make_async_copy(buf_o, s_sh.at[idx_r[0]], sem_s).wait()

        def load_idx(u, q, sp):
            ebase = (u * NW + wid) * C
            pltpu.async_copy(row_hbm.at[pl.ds(ebase, C)], idx_r[q],
                             sem_ir[sp])
            pltpu.async_copy(col_hbm.at[pl.ds(ebase, C)], idx_c[q],
                             sem_ic[sp])

        def wait_idx(q, sp):
            pltpu.make_async_copy(row_hbm.at[pl.ds(0, C)], idx_r[q],
                                  sem_ir[sp]).wait()
            pltpu.make_async_copy(col_hbm.at[pl.ds(0, C)], idx_c[q],
                                  sem_ic[sp]).wait()

        def issue_gathers(u, q, p2):
            pltpu.async_copy(a_hbm.at[idx_r[q]], buf_a[p2], sem_a)
            pltpu.async_copy(b_hbm.at[idx_c[q]], buf_b[p2], sem_b)
            ebase = (u * NW + wid) * C
            pltpu.async_copy(e_hbm.at[pl.ds(ebase, C)], buf_e[p2], sem_e)

        def wait_gathers(q, p2):
            pltpu.make_async_copy(a_hbm.at[idx_r[q]], buf_a[p2], sem_a).wait()
            pltpu.make_async_copy(b_hbm.at[idx_c[q]], buf_b[p2], sem_b).wait()
            pltpu.make_async_copy(e_hbm.at[pl.ds(0, C)], buf_e[p2],
                                  sem_e).wait()

        load_idx(0, 0, 0)
        load_idx(1, 1, 1)
        wait_idx(0, 0)
        issue_gathers(0, 0, 0)

        def chunk_step(t, par):
            p2 = par % 2
            wait_gathers(par, p2)

            @pl.when(t > 0)
            def _():
                wait_scatter()

            @pl.when(t < NT - 2)
            def _():
                load_idx(t + 2, (par + 2) % 4, par % 2)

            @pl.when(t < NT - 1)
            def _():
                wait_idx((par + 1) % 4, (par + 1) % 2)
                issue_gathers(t + 1, (par + 1) % 4, 1 - p2)

            a_b, b_b, e_b = buf_a[p2], buf_b[p2], buf_e[p2]

            def comp(r, _):
                for k8 in range(H // 16):
                    sl = pl.ds(k8 * 16, 16)
                    v = a_b[r, sl] + b_b[r, sl] + e_b[r, sl]
                    buf_o[r, sl] = jnp.maximum(v, 0.0)
                return 0

            lax.fori_loop(0, C, comp, 0)
            pltpu.async_copy(buf_o, s_sh.at[idx_r[par]], sem_s, add=True)

        def quad_body(j, _):
            for par in range(4):
                chunk_step(j * 4 + par, par)
            return 0

        lax.fori_loop(0, NT // 4, quad_body, 0)
        for tail in range(NT - NT % 4, NT):
            chunk_step(jnp.int32(tail), tail % 4)
        wait_scatter()
        plsc.subcore_barrier()

        obase = cid * NP + rbase
        for t in range(RPS // 40):
            pltpu.sync_copy(s_sh.at[pl.ds(rbase + t * 40, 40)],
                            out_hbm.at[pl.ds(obase + t * 40, 40)])

    return k


def _make_deg_sc():
    @functools.partial(
        pl.kernel,
        mesh=_sc_mesh(),
        out_type=jax.ShapeDtypeStruct((2 * NP, H), jnp.float32),
        scratch_types=[
            pltpu.VMEM_SHARED((NP, H), jnp.float32),
            [pltpu.VMEM((CD,), jnp.int32)] * 2,
            pltpu.VMEM((CD, H), jnp.float32),
            pltpu.SemaphoreType.DMA,
        ],
        interpret=_INTERPRET,
    )
    def k(row_hbm, out_hbm, s_sh, idx_r, ones_b, sem_s):
        cid = lax.axis_index("c")
        sid = lax.axis_index("s")
        wid = sid * NC + cid

        _zero_f32(ones_b, CD)
        rbase = sid * RPS
        for t in range(RPS // CD):
            pltpu.sync_copy(ones_b, s_sh.at[pl.ds(rbase + t * CD, CD)])

        def obody(r, _):
            for k8 in range(H // 16):
                ones_b[r, pl.ds(k8 * 16, 16)] = jnp.ones((16,), jnp.float32)
            return 0
        lax.fori_loop(0, CD, obody, 0)
        plsc.subcore_barrier()

        def wait_scatter():
            pltpu.make_async_copy(ones_b, s_sh.at[idx_r[0]], sem_s).wait()

        def do_chunk(t, par):
            ebase = (t * NW + wid) * CD
            pltpu.sync_copy(row_hbm.at[pl.ds(ebase, CD)], idx_r[par])

            @pl.when(t > 1)
            def _():
                wait_scatter()
            pltpu.async_copy(ones_b, s_sh.at[idx_r[par]], sem_s, add=True)

        def pair_body(j, _):
            for par in range(2):
                do_chunk(j * 2 + par, par)
            return 0

        lax.fori_loop(0, NTD // 2, pair_body, 0)
        do_chunk(NTD - 1, (NTD - 1) % 2)
        wait_scatter()
        wait_scatter()
        plsc.subcore_barrier()

        obase = cid * NP + rbase
        for t in range(RPS // CD):
            pltpu.sync_copy(s_sh.at[pl.ds(rbase + t * CD, CD)],
                            out_hbm.at[pl.ds(obase + t * CD, CD)])

    return k


def _pad8(v):
    return jnp.broadcast_to(v[None, :], (8, H))


def kernel(x, edge_index, edge_attr, batch, params):
    row = edge_index[0]
    col = edge_index[1]
    layers = params["layers"]

    w1a = [p["mW1"][:H] for p in layers]
    w1b = [p["mW1"][H:2 * H] for p in layers]
    wea = jnp.concatenate([p["mW1"][2 * H:] for p in layers] +
                          [jnp.zeros((4, H), jnp.float32)], axis=0)
    mb1s = jnp.concatenate(
        [jnp.stack([p["mb1"] for p in layers]),
         jnp.zeros((5, H), jnp.float32)], axis=0)
    vecs = [jnp.concatenate(
        [jnp.stack([p["mb2"], p["ub1"], p["ub2"], p["ln_g"], p["ln_b"]]),
         jnp.zeros((3, H), jnp.float32)], axis=0) for p in layers]
    uw1a = [p["uW1"][:H] for p in layers]
    uw1b = [p["uW1"][H:] for p in layers]
    x_pad = jnp.pad(x, ((0, NP - N), (0, 0)))
    batch_pad = jnp.pad(batch.astype(jnp.int32), (0, NP - N),
                        constant_values=127)

    h, a_cur, b_cur = _in_proj(x_pad, params["in_W"], _pad8(params["in_b"]),
                               w1a[0], w1b[0])
    ea1, ea2, ea3 = _edge_pre(edge_attr, wea, mb1s)
    eas = [ea1, ea2, ea3]

    sc_edge = _make_edge_sc()
    deg_all = _make_deg_sc()(row)
    d0, d1 = deg_all[:NP], deg_all[NP:]

    for i in range(3):
        s_all = sc_edge(a_cur, b_cur, eas[i], row, col)
        s0, s1 = s_all[:NP], s_all[NP:]
        p = layers[i]
        if i < 2:
            h, a_cur, b_cur = _update(
                i == 0, True, h, s0, s1, d0, d1,
                p["mW2"], uw1a[i], uw1b[i], p["uW2"], vecs[i],
                w1a[i + 1], w1b[i + 1])
        else:
            bid_rep = jnp.broadcast_to(batch_pad[:, None], (NP, H))
            out = _update(
                False, False, h, s0, s1, d0, d1,
                p["mW2"], uw1a[i], uw1b[i], p["uW2"], vecs[i],
                bid_rep=bid_rep, out_w=params["out_W"],
                out_b8=_pad8(params["out_b"]))

    return out
